# initial kernel scaffold (unmeasured)
import jax
import jax.numpy as jnp
from jax import lax
from jax.experimental import pallas as pl
from jax.experimental.pallas import tpu as pltpu

N_DEV = 16


def kernel(x, Win0, Wout0, Win1, Wout1, Win2, Wout2):
    b, d_sh = x.shape
    h = Win0.shape[1]
    rows = b // N_DEV

    def body(x_ref, win0_ref, wout0_ref, win1_ref, wout1_ref, win2_ref,
             wout2_ref, out_ref, partial_ref, recva_ref, hfull_ref,
             senda_sems, recva_sems, sendb_sems, recvb_sems):
        my = lax.axis_index("i")
        wins = [win0_ref, win1_ref, win2_ref]
        wouts = [wout0_ref, wout1_ref, wout2_ref]

        x_cur = x_ref[:, :]
        for layer in range(3):
            partial_ref[:, :] = jnp.dot(
                x_cur, wins[layer][:, :], preferred_element_type=jnp.float32
            )

            sends_a = []
            for o in range(1, N_DEV):
                tgt = lax.rem(my + o, N_DEV)
                rdma = pltpu.make_async_remote_copy(
                    src_ref=partial_ref.at[pl.ds(tgt * rows, rows)],
                    dst_ref=recva_ref.at[N_DEV - o],
                    send_sem=senda_sems.at[o],
                    recv_sem=recva_sems.at[N_DEV - o],
                    device_id=(tgt,),
                    device_id_type=pl.DeviceIdType.MESH,
                )
                rdma.start()
                sends_a.append(rdma)

            acc = partial_ref[pl.ds(my * rows, rows), :]
            for o in range(1, N_DEV):
                recv = pltpu.make_async_remote_copy(
                    src_ref=recva_ref.at[o],
                    dst_ref=recva_ref.at[o],
                    send_sem=senda_sems.at[o],
                    recv_sem=recva_sems.at[o],
                    device_id=(my,),
                    device_id_type=pl.DeviceIdType.MESH,
                )
                recv.wait_recv()
                acc = acc + recva_ref[o, :, :]
            for r in sends_a:
                r.wait_send()

            hfull_ref[pl.ds(my * rows, rows), :] = jnp.maximum(acc, 0.0)

            sends_b = []
            for o in range(1, N_DEV):
                tgt = lax.rem(my + o, N_DEV)
                rdma = pltpu.make_async_remote_copy(
                    src_ref=hfull_ref.at[pl.ds(my * rows, rows)],
                    dst_ref=hfull_ref.at[pl.ds(my * rows, rows)],
                    send_sem=sendb_sems.at[o],
                    recv_sem=recvb_sems.at[N_DEV - o],
                    device_id=(tgt,),
                    device_id_type=pl.DeviceIdType.MESH,
                )
                rdma.start()
                sends_b.append(rdma)
            for o in range(1, N_DEV):
                src_dev = lax.rem(my + o, N_DEV)
                recv = pltpu.make_async_remote_copy(
                    src_ref=hfull_ref.at[pl.ds(src_dev * rows, rows)],
                    dst_ref=hfull_ref.at[pl.ds(src_dev * rows, rows)],
                    send_sem=sendb_sems.at[o],
                    recv_sem=recvb_sems.at[o],
                    device_id=(my,),
                    device_id_type=pl.DeviceIdType.MESH,
                )
                recv.wait_recv()
            for r in sends_b:
                r.wait_send()

            x_cur = jnp.dot(
                hfull_ref[:, :], wouts[layer][:, :],
                preferred_element_type=jnp.float32,
            )

        out_ref[:, :] = x_cur

    return pl.pallas_call(
        body,
        out_shape=jax.ShapeDtypeStruct((b, d_sh), jnp.float32),
        in_specs=[pl.BlockSpec(memory_space=pltpu.VMEM)] * 7,
        out_specs=pl.BlockSpec(memory_space=pltpu.VMEM),
        scratch_shapes=[
            pltpu.VMEM((b, h), jnp.float32),
            pltpu.VMEM((N_DEV, rows, h), jnp.float32),
            pltpu.VMEM((b, h), jnp.float32),
            pltpu.SemaphoreType.DMA((N_DEV,)),
            pltpu.SemaphoreType.DMA((N_DEV,)),
            pltpu.SemaphoreType.DMA((N_DEV,)),
            pltpu.SemaphoreType.DMA((N_DEV,)),
        ],
        compiler_params=pltpu.CompilerParams(collective_id=0),
    )(x, Win0, Wout0, Win1, Wout1, Win2, Wout2)


# baseline (device time: 93762 ns/iter reference)
import jax
import jax.numpy as jnp
from jax import lax
from jax.experimental import pallas as pl
from jax.experimental.pallas import tpu as pltpu

N_DEV = 16


def kernel(x, Win0, Wout0, Win1, Wout1, Win2, Wout2):
    b, d_sh = x.shape
    h = Win0.shape[1]
    rows = b // N_DEV

    def body(x_ref, win0_ref, wout0_ref, win1_ref, wout1_ref, win2_ref,
             wout2_ref, out_ref, partial_ref, recva_ref, hfull_ref,
             senda_sems, recva_sems, sendb_sems, recvb_sems):
        my = lax.axis_index("i")
        wins = [win0_ref, win1_ref, win2_ref]
        wouts = [wout0_ref, wout1_ref, wout2_ref]

        x_cur = x_ref[:, :]
        for layer in range(3):
            partial_ref[:, :] = jnp.dot(
                x_cur, wins[layer][:, :], preferred_element_type=jnp.float32
            )

            sends_a = []
            for o in range(1, N_DEV):
                tgt = lax.rem(my + o, N_DEV)
                rdma = pltpu.make_async_remote_copy(
                    src_ref=partial_ref.at[pl.ds(tgt * rows, rows)],
                    dst_ref=recva_ref.at[N_DEV - o],
                    send_sem=senda_sems.at[o],
                    recv_sem=recva_sems.at[N_DEV - o],
                    device_id=(tgt,),
                    device_id_type=pl.DeviceIdType.MESH,
                )
                rdma.start()
                sends_a.append(rdma)

            acc = partial_ref[pl.ds(my * rows, rows), :]
            for o in range(1, N_DEV):
                recv = pltpu.make_async_remote_copy(
                    src_ref=recva_ref.at[o],
                    dst_ref=recva_ref.at[o],
                    send_sem=senda_sems.at[o],
                    recv_sem=recva_sems.at[o],
                    device_id=(my,),
                    device_id_type=pl.DeviceIdType.MESH,
                )
                recv.wait_recv()
                acc = acc + recva_ref[o, :, :]
            for r in sends_a:
                r.wait_send()

            hfull_ref[pl.ds(my * rows, rows), :] = jnp.maximum(acc, 0.0)

            sends_b = []
            for o in range(1, N_DEV):
                tgt = lax.rem(my + o, N_DEV)
                rdma = pltpu.make_async_remote_copy(
                    src_ref=hfull_ref.at[pl.ds(my * rows, rows)],
                    dst_ref=hfull_ref.at[pl.ds(my * rows, rows)],
                    send_sem=sendb_sems.at[o],
                    recv_sem=recvb_sems.at[N_DEV - o],
                    device_id=(tgt,),
                    device_id_type=pl.DeviceIdType.MESH,
                )
                rdma.start()
                sends_b.append(rdma)
            for o in range(1, N_DEV):
                src_dev = lax.rem(my + o, N_DEV)
                recv = pltpu.make_async_remote_copy(
                    src_ref=hfull_ref.at[pl.ds(src_dev * rows, rows)],
                    dst_ref=hfull_ref.at[pl.ds(src_dev * rows, rows)],
                    send_sem=sendb_sems.at[o],
                    recv_sem=recvb_sems.at[o],
                    device_id=(my,),
                    device_id_type=pl.DeviceIdType.MESH,
                )
                recv.wait_recv()
            for r in sends_b:
                r.wait_send()

            x_cur = jnp.dot(
                hfull_ref[:, :], wouts[layer][:, :],
                preferred_element_type=jnp.float32,
            )

        out_ref[:, :] = x_cur

    return pl.pallas_call(
        body,
        out_shape=jax.ShapeDtypeStruct((b, d_sh), jnp.float32),
        in_specs=[pl.BlockSpec(memory_space=pltpu.VMEM)] * 7,
        out_specs=pl.BlockSpec(memory_space=pltpu.VMEM),
        scratch_shapes=[
            pltpu.VMEM((b, h), jnp.float32),
            pltpu.VMEM((N_DEV, rows, h), jnp.float32),
            pltpu.VMEM((b, h), jnp.float32),
            pltpu.SemaphoreType.DMA((N_DEV,)),
            pltpu.SemaphoreType.DMA((N_DEV,)),
            pltpu.SemaphoreType.DMA((N_DEV,)),
            pltpu.SemaphoreType.DMA((N_DEV,)),
        ],
    )(x, Win0, Wout0, Win1, Wout1, Win2, Wout2)


# device time: 64434 ns/iter; 1.4552x vs baseline; 1.4552x over previous
import jax
import jax.numpy as jnp
from jax import lax
from jax.experimental import pallas as pl
from jax.experimental.pallas import tpu as pltpu

N_DEV = 16


def kernel(x, Win0, Wout0, Win1, Wout1, Win2, Wout2):
    b, d_sh = x.shape
    h = Win0.shape[1]
    rows = b // N_DEV

    def body(x_ref, win0_ref, wout0_ref, win1_ref, wout1_ref, win2_ref,
             wout2_ref, out_ref, partial_ref, recva_ref, hfull_ref,
             senda_sems, recva_sems, sendb_sems, recvb_sems):
        my = lax.axis_index("i")
        wins = [win0_ref, win1_ref, win2_ref]
        wouts = [wout0_ref, wout1_ref, wout2_ref]

        x_cur = x_ref[:, :]
        for layer in range(3):
            partial_ref[:, :] = jnp.dot(
                x_cur, wins[layer][:, :], preferred_element_type=jnp.float32
            ).astype(jnp.bfloat16)

            sends_a = []
            for o in range(1, N_DEV):
                tgt = lax.rem(my + o, N_DEV)
                rdma = pltpu.make_async_remote_copy(
                    src_ref=partial_ref.at[pl.ds(tgt * rows, rows)],
                    dst_ref=recva_ref.at[N_DEV - o],
                    send_sem=senda_sems.at[o],
                    recv_sem=recva_sems.at[N_DEV - o],
                    device_id=(tgt,),
                    device_id_type=pl.DeviceIdType.MESH,
                )
                rdma.start()
                sends_a.append(rdma)

            acc = partial_ref[pl.ds(my * rows, rows), :].astype(jnp.float32)
            for o in range(1, N_DEV):
                recv = pltpu.make_async_remote_copy(
                    src_ref=recva_ref.at[o],
                    dst_ref=recva_ref.at[o],
                    send_sem=senda_sems.at[o],
                    recv_sem=recva_sems.at[o],
                    device_id=(my,),
                    device_id_type=pl.DeviceIdType.MESH,
                )
                recv.wait_recv()
                acc = acc + recva_ref[o, :, :].astype(jnp.float32)
            for r in sends_a:
                r.wait_send()

            hfull_ref[pl.ds(my * rows, rows), :] = jnp.maximum(acc, 0.0).astype(
                jnp.bfloat16
            )

            sends_b = []
            for o in range(1, N_DEV):
                tgt = lax.rem(my + o, N_DEV)
                rdma = pltpu.make_async_remote_copy(
                    src_ref=hfull_ref.at[pl.ds(my * rows, rows)],
                    dst_ref=hfull_ref.at[pl.ds(my * rows, rows)],
                    send_sem=sendb_sems.at[o],
                    recv_sem=recvb_sems.at[N_DEV - o],
                    device_id=(tgt,),
                    device_id_type=pl.DeviceIdType.MESH,
                )
                rdma.start()
                sends_b.append(rdma)
            for o in range(1, N_DEV):
                src_dev = lax.rem(my + o, N_DEV)
                recv = pltpu.make_async_remote_copy(
                    src_ref=hfull_ref.at[pl.ds(src_dev * rows, rows)],
                    dst_ref=hfull_ref.at[pl.ds(src_dev * rows, rows)],
                    send_sem=sendb_sems.at[o],
                    recv_sem=recvb_sems.at[o],
                    device_id=(my,),
                    device_id_type=pl.DeviceIdType.MESH,
                )
                recv.wait_recv()
            for r in sends_b:
                r.wait_send()

            x_cur = jnp.dot(
                hfull_ref[:, :].astype(jnp.float32), wouts[layer][:, :],
                preferred_element_type=jnp.float32,
            )

        out_ref[:, :] = x_cur

    return pl.pallas_call(
        body,
        out_shape=jax.ShapeDtypeStruct((b, d_sh), jnp.float32),
        in_specs=[pl.BlockSpec(memory_space=pltpu.VMEM)] * 7,
        out_specs=pl.BlockSpec(memory_space=pltpu.VMEM),
        scratch_shapes=[
            pltpu.VMEM((b, h), jnp.bfloat16),
            pltpu.VMEM((N_DEV, rows, h), jnp.bfloat16),
            pltpu.VMEM((b, h), jnp.bfloat16),
            pltpu.SemaphoreType.DMA((N_DEV,)),
            pltpu.SemaphoreType.DMA((N_DEV,)),
            pltpu.SemaphoreType.DMA((N_DEV,)),
            pltpu.SemaphoreType.DMA((N_DEV,)),
        ],
    )(x, Win0, Wout0, Win1, Wout1, Win2, Wout2)
